# Initial kernel scaffold; baseline (speedup 1.0000x reference)
import jax
import jax.numpy as jnp
from jax.experimental import pallas as pl


def kernel(vmem, labels):
    raise NotImplementedError("write your pallas kernel here")


# speedup vs baseline: 89.3394x; 89.3394x over previous
import functools

import jax
import jax.numpy as jnp
from jax import lax
from jax.experimental import pallas as pl
from jax.experimental.pallas import tpu as pltpu
from jax.experimental.pallas import tpu_sc as plsc

THRESH = 1.0
GAP = 5
T = 512
L = 16
NC = 2
NS = 16
NW = NC * NS
ROWS = 32768
ROWS_PER_W = ROWS // NW
RB = 32
NBLK = ROWS_PER_W // RB
NCH = T // L
BIG = 1 << 22

_GDN = lax.GatherDimensionNumbers(
    offset_dims=(), collapsed_slice_dims=(0,), start_index_map=(0,)
)


def _vgather(x, idx):
    return lax.gather(
        x,
        idx[:, None],
        _GDN,
        slice_sizes=(1,),
        mode=lax.GatherScatterMode.PROMISE_IN_BOUNDS,
    )


def _splat_last(x):
    return _vgather(x, jnp.full((L,), L - 1, jnp.int32))


def _sc_body(v_hbm, lab_hbm, out_hbm, rows_v, posbuf, labels_v, accbuf):
    wid = lax.axis_index("s") * NC + lax.axis_index("c")
    wstart = wid * ROWS_PER_W
    iota = lax.iota(jnp.int32, L)
    lane0 = iota == 0
    shl = jnp.maximum(iota - 1, 0)

    pltpu.sync_copy(lab_hbm.at[pl.ds(wstart, ROWS_PER_W)], labels_v)

    def row_body(r, rbase, acc):

        def p1(j, c):
            cnt, vmx = c
            v_c = rows_v[r, pl.ds(j * L, L)]
            m = v_c >= THRESH
            loc = plsc.cumsum(m.astype(jnp.int32))
            s_c = loc + cnt
            gi = j * L + iota
            plsc.store_scatter(posbuf, [s_c - 1], gi, mask=m)
            cnt = cnt + plsc.all_reduce_population_count(m)
            vmx = jnp.maximum(vmx, v_c)
            return cnt, vmx

        cnt0 = jnp.zeros((L,), jnp.int32)
        vmx0 = jnp.full((L,), -jnp.inf, jnp.float32)
        n_splat, vmax_vec = lax.fori_loop(0, NCH, p1, (cnt0, vmx0))
        n_s = jnp.max(n_splat)

        def p2(j, c):
            si, prevlast, mink, lp = c
            pos_c = posbuf[pl.ds(j * L, L)]
            gi = j * L + iota
            validm = gi < n_splat
            prev = jnp.where(lane0, prevlast, _vgather(pos_c, shl))
            gap = pos_c - prev
            newflag = (gap > GAP) & (gi > 0) & validm
            startval = jnp.where(newflag | (gi == 0), gi, 0)
            incl = jnp.maximum(plsc.cummax(startval), si)
            excl = jnp.where(lane0, si, _vgather(incl, shl))
            key = jnp.where(newflag, (gi - excl) * 1024 + prev, BIG)
            mink = jnp.minimum(mink, key)
            lp = jnp.maximum(lp, jnp.where(validm, pos_c, 0))
            return _splat_last(incl), _splat_last(pos_c), mink, lp

        nch = (n_s + (L - 1)) // L
        si0 = jnp.zeros((L,), jnp.int32)
        mink0 = jnp.full((L,), BIG, jnp.int32)
        si_f, _, mink_vec, lp_vec = lax.fori_loop(
            0, nch, p2, (si0, si0, mink0, si0)
        )

        si_s = jnp.max(si_f)
        lp_s = jnp.max(lp_vec)
        mk = jnp.min(mink_vec)
        keyf = (n_s - si_s) * 1024 + lp_s
        mk = jnp.minimum(mk, jnp.where(n_s > 0, keyf, BIG))
        tstar = jnp.bitwise_and(mk, 1023)

        vat = plsc.load_gather(
            rows_v, [jnp.full((L,), r, jnp.int32), jnp.full((L,), tstar, jnp.int32)]
        )
        vmax_s = jnp.max(vmax_vec)

        g = rbase + r
        labv = labels_v[pl.ds((g // L) * L, L)]
        lab = _vgather(labv, jnp.full((L,), g % L, jnp.int32))

        has = n_splat > 0
        c_fp = lane0 & (lab == 0) & has
        c_miss = lane0 & (lab == 1) & jnp.logical_not(has)
        acc = acc + jnp.where(c_fp, vat - THRESH, 0.0)
        acc = acc + jnp.where(c_miss, THRESH - vmax_s, 0.0)
        return acc

    def blk_body(b, acc):
        pltpu.sync_copy(v_hbm.at[pl.ds(wstart + b * RB, RB)], rows_v)
        rbase = b * RB

        def rloop(r, a):
            return row_body(r, rbase, a)

        return lax.fori_loop(0, RB, rloop, acc)

    acc = lax.fori_loop(0, NBLK, blk_body, jnp.zeros((L,), jnp.float32))
    accbuf[...] = acc
    pltpu.sync_copy(accbuf, out_hbm.at[wid])


@functools.partial(
    pl.kernel,
    out_type=jax.ShapeDtypeStruct((NW, L), jnp.float32),
    compiler_params=pltpu.CompilerParams(needs_layout_passes=False),
    mesh=plsc.VectorSubcoreMesh(core_axis_name="c", subcore_axis_name="s"),
    scratch_types=[
        pltpu.VMEM((RB, T), jnp.float32),
        pltpu.VMEM((T,), jnp.int32),
        pltpu.VMEM((ROWS_PER_W,), jnp.int32),
        pltpu.VMEM((L,), jnp.float32),
    ],
)
def _stca_loss_sc(v_hbm, lab_hbm, out_hbm, rows_v, posbuf, labels_v, accbuf):
    _sc_body(v_hbm, lab_hbm, out_hbm, rows_v, posbuf, labels_v, accbuf)


def kernel(vmem, labels):
    B, N, Tdim = vmem.shape
    v2 = vmem.reshape(B * N, Tdim)
    lab = labels.reshape(B * N).astype(jnp.int32)
    partials = _stca_loss_sc(v2, lab)
    return jnp.sum(partials)


# speedup vs baseline: 120.8580x; 1.3528x over previous
import functools

import jax
import jax.numpy as jnp
from jax import lax
from jax.experimental import pallas as pl
from jax.experimental.pallas import tpu as pltpu
from jax.experimental.pallas import tpu_sc as plsc

THRESH = 1.0
GAP = 5
T = 512
L = 16
NC = 2
NS = 16
NW = NC * NS
ROWS = 32768
ROWS_PER_W = ROWS // NW
RB = 32
NBLK = ROWS_PER_W // RB
NCH = T // L
BIG = 1 << 22

_GDN = lax.GatherDimensionNumbers(
    offset_dims=(), collapsed_slice_dims=(0,), start_index_map=(0,)
)


def _vgather(x, idx):
    return lax.gather(
        x,
        idx[:, None],
        _GDN,
        slice_sizes=(1,),
        mode=lax.GatherScatterMode.PROMISE_IN_BOUNDS,
    )


def _splat_last(x):
    return _vgather(x, jnp.full((L,), L - 1, jnp.int32))


def _sc_body(v_hbm, lab_hbm, out_hbm, rows_v, posbuf, labels_v, accbuf):
    wid = lax.axis_index("s") * NC + lax.axis_index("c")
    wstart = wid * ROWS_PER_W
    iota = lax.iota(jnp.int32, L)
    lane0 = iota == 0
    shl = jnp.maximum(iota - 1, 0)

    pltpu.sync_copy(lab_hbm.at[pl.ds(wstart, ROWS_PER_W)], labels_v)

    def row_body(r, rbase, acc):
        g = rbase + r
        labv = labels_v[pl.ds((g // L) * L, L)]
        lab_vec = _vgather(labv, jnp.full((L,), g % L, jnp.int32))
        lab_s = lab_vec[0]

        def miss_branch(_):
            def p1(j, vmx):
                return jnp.maximum(vmx, rows_v[r, pl.ds(j * L, L)])

            vmx = lax.fori_loop(
                0, NCH, p1, jnp.full((L,), -jnp.inf, jnp.float32)
            )
            vmax_s = jnp.max(vmx)
            hit = lane0 & (lab_vec == 1) & (vmax_s < THRESH)
            return jnp.where(hit, THRESH - vmax_s, 0.0)

        def cluster_branch(_):

            def p1(j, cnt):
                v_c = rows_v[r, pl.ds(j * L, L)]
                m = v_c >= THRESH
                s_c = plsc.cumsum(m.astype(jnp.int32)) + cnt
                gi = j * L + iota
                plsc.store_scatter(posbuf, [s_c - 1], gi, mask=m)
                return cnt + plsc.all_reduce_population_count(m)

            n_splat = lax.fori_loop(0, NCH, p1, jnp.zeros((L,), jnp.int32))
            n_s = n_splat[0]

            def p2(j, c):
                si, prevlast, _, mink = c
                pos_c = posbuf[pl.ds(j * L, L)]
                gi = j * L + iota
                validm = gi < n_splat
                prev = jnp.where(lane0, prevlast, _vgather(pos_c, shl))
                gap = pos_c - prev
                newflag = (gap > GAP) & (gi > 0) & validm
                startval = jnp.where(newflag | (gi == 0), gi, 0)
                incl = jnp.maximum(plsc.cummax(startval), si)
                excl = jnp.where(lane0, si, _vgather(incl, shl))
                key = jnp.where(newflag, (gi - excl) * 1024 + prev, BIG)
                mink = jnp.minimum(mink, key)
                return _splat_last(incl), _splat_last(pos_c), pos_c, mink

            nch = (n_s + (L - 1)) // L
            si0 = jnp.zeros((L,), jnp.int32)
            mink0 = jnp.full((L,), BIG, jnp.int32)
            si_f, _, lastp, mink_vec = lax.fori_loop(
                0, nch, p2, (si0, si0, si0, mink0)
            )

            lanes_last = jnp.bitwise_and(n_s - 1, L - 1)
            lp_s = _vgather(lastp, jnp.full((L,), lanes_last, jnp.int32))[0]
            mk = jnp.min(mink_vec)
            keyf = (n_s - si_f[0]) * 1024 + lp_s
            mk = jnp.minimum(mk, jnp.where(n_s > 0, keyf, BIG))
            tstar = jnp.bitwise_and(mk, 1023)

            vat = plsc.load_gather(
                rows_v,
                [jnp.full((L,), r, jnp.int32), jnp.full((L,), tstar, jnp.int32)],
            )
            return jnp.where(lane0 & (n_splat > 0), vat - THRESH, 0.0)

        contrib = lax.cond(lab_s == 0, cluster_branch, miss_branch, 0)
        return acc + contrib

    def blk_body(b, acc):
        pltpu.sync_copy(v_hbm.at[pl.ds(wstart + b * RB, RB)], rows_v)
        rbase = b * RB

        def rloop(r, a):
            return row_body(r, rbase, a)

        return lax.fori_loop(0, RB, rloop, acc)

    acc = lax.fori_loop(0, NBLK, blk_body, jnp.zeros((L,), jnp.float32))
    accbuf[...] = acc
    pltpu.sync_copy(accbuf, out_hbm.at[wid])


@functools.partial(
    pl.kernel,
    out_type=jax.ShapeDtypeStruct((NW, L), jnp.float32),
    compiler_params=pltpu.CompilerParams(needs_layout_passes=False),
    mesh=plsc.VectorSubcoreMesh(core_axis_name="c", subcore_axis_name="s"),
    scratch_types=[
        pltpu.VMEM((RB, T), jnp.float32),
        pltpu.VMEM((T,), jnp.int32),
        pltpu.VMEM((ROWS_PER_W,), jnp.int32),
        pltpu.VMEM((L,), jnp.float32),
    ],
)
def _stca_loss_sc(v_hbm, lab_hbm, out_hbm, rows_v, posbuf, labels_v, accbuf):
    _sc_body(v_hbm, lab_hbm, out_hbm, rows_v, posbuf, labels_v, accbuf)


def kernel(vmem, labels):
    B, N, Tdim = vmem.shape
    v2 = vmem.reshape(B * N, Tdim)
    lab = labels.reshape(B * N).astype(jnp.int32)
    partials = _stca_loss_sc(v2, lab)
    return jnp.sum(partials)


# speedup vs baseline: 256.0158x; 2.1183x over previous
import functools

import jax
import jax.numpy as jnp
from jax import lax
from jax.experimental import pallas as pl
from jax.experimental.pallas import tpu as pltpu
from jax.experimental.pallas import tpu_sc as plsc

THRESH = 1.0
GAP = 5
T = 512
L = 16
NC = 2
NS = 16
NW = NC * NS
ROWS = 32768
ROWS_PER_W = ROWS // NW
RB = 32
NBLK = ROWS_PER_W // RB
NCH = T // L
BIG = 1 << 22

_GDN = lax.GatherDimensionNumbers(
    offset_dims=(), collapsed_slice_dims=(0,), start_index_map=(0,)
)


def _vgather(x, idx):
    return lax.gather(
        x,
        idx[:, None],
        _GDN,
        slice_sizes=(1,),
        mode=lax.GatherScatterMode.PROMISE_IN_BOUNDS,
    )


def _splat_last(x):
    return _vgather(x, jnp.full((L,), L - 1, jnp.int32))


def _sc_body(v_hbm, lab_hbm, out_hbm, rows_v, posbuf, labels_v, accbuf):
    wid = lax.axis_index("s") * NC + lax.axis_index("c")
    wstart = wid * ROWS_PER_W
    iota = lax.iota(jnp.int32, L)
    lane0 = iota == 0
    shl = jnp.maximum(iota - 1, 0)

    pltpu.sync_copy(lab_hbm.at[pl.ds(wstart, ROWS_PER_W)], labels_v)

    def row_body(r, rbase, acc):
        g = rbase + r
        labv = labels_v[pl.ds((g // L) * L, L)]
        lab_vec = _vgather(labv, jnp.full((L,), g % L, jnp.int32))
        lab_s = lab_vec[0]

        def miss_branch(_):
            @plsc.parallel_loop(
                0, NCH, carry=jnp.full((L,), -jnp.inf, jnp.float32), unroll=4
            )
            def vmx(j, c):
                return jnp.maximum(c, rows_v[r, pl.ds(j * L, L)])

            vmax_s = jnp.max(vmx)
            hit = lane0 & (lab_vec == 1) & (vmax_s < THRESH)
            return jnp.where(hit, THRESH - vmax_s, 0.0)

        def cluster_branch(_):

            @plsc.parallel_loop(
                0, NCH, carry=jnp.zeros((L,), jnp.int32), unroll=2
            )
            def n_splat(j, cnt):
                v_c = rows_v[r, pl.ds(j * L, L)]
                m = v_c >= THRESH
                s_c = plsc.cumsum(m.astype(jnp.int32)) + cnt
                gi = j * L + iota
                plsc.store_scatter(posbuf, [s_c - 1], gi, mask=m)
                return cnt + plsc.all_reduce_population_count(m)

            n_s = n_splat[0]

            def p2(j, c):
                si, prevlast, _, mink = c
                pos_c = posbuf[pl.ds(j * L, L)]
                gi = j * L + iota
                validm = gi < n_splat
                prev = jnp.where(lane0, prevlast, _vgather(pos_c, shl))
                gap = pos_c - prev
                newflag = (gap > GAP) & (gi > 0) & validm
                startval = jnp.where(newflag | (gi == 0), gi, 0)
                incl_loc = plsc.cummax(startval)
                excl = jnp.where(
                    lane0, si, jnp.maximum(_vgather(incl_loc, shl), si)
                )
                key = jnp.where(newflag, (gi - excl) * 1024 + prev, BIG)
                mink = jnp.minimum(mink, key)
                si = jnp.maximum(si, _splat_last(incl_loc))
                return si, _splat_last(pos_c), pos_c, mink

            nch = (n_s + (L - 1)) // L
            si0 = jnp.zeros((L,), jnp.int32)
            mink0 = jnp.full((L,), BIG, jnp.int32)
            si_f, _, lastp, mink_vec = lax.fori_loop(
                0, nch, p2, (si0, si0, si0, mink0)
            )

            lanes_last = jnp.bitwise_and(n_s - 1, L - 1)
            lp_s = _vgather(lastp, jnp.full((L,), lanes_last, jnp.int32))[0]
            mk = jnp.min(mink_vec)
            keyf = (n_s - si_f[0]) * 1024 + lp_s
            mk = jnp.minimum(mk, jnp.where(n_s > 0, keyf, BIG))
            tstar = jnp.bitwise_and(mk, 1023)

            vat = plsc.load_gather(
                rows_v,
                [jnp.full((L,), r, jnp.int32), jnp.full((L,), tstar, jnp.int32)],
            )
            return jnp.where(lane0 & (n_splat > 0), vat - THRESH, 0.0)

        contrib = lax.cond(lab_s == 0, cluster_branch, miss_branch, 0)
        return acc + contrib

    def blk_body(b, acc):
        pltpu.sync_copy(v_hbm.at[pl.ds(wstart + b * RB, RB)], rows_v)
        rbase = b * RB

        def rloop(r, a):
            return row_body(r, rbase, a)

        return lax.fori_loop(0, RB, rloop, acc)

    acc = lax.fori_loop(0, NBLK, blk_body, jnp.zeros((L,), jnp.float32))
    accbuf[...] = acc
    pltpu.sync_copy(accbuf, out_hbm.at[wid])


@functools.partial(
    pl.kernel,
    out_type=jax.ShapeDtypeStruct((NW, L), jnp.float32),
    compiler_params=pltpu.CompilerParams(needs_layout_passes=False),
    mesh=plsc.VectorSubcoreMesh(core_axis_name="c", subcore_axis_name="s"),
    scratch_types=[
        pltpu.VMEM((RB, T), jnp.float32),
        pltpu.VMEM((T,), jnp.int32),
        pltpu.VMEM((ROWS_PER_W,), jnp.int32),
        pltpu.VMEM((L,), jnp.float32),
    ],
)
def _stca_loss_sc(v_hbm, lab_hbm, out_hbm, rows_v, posbuf, labels_v, accbuf):
    _sc_body(v_hbm, lab_hbm, out_hbm, rows_v, posbuf, labels_v, accbuf)


def kernel(vmem, labels):
    B, N, Tdim = vmem.shape
    v2 = vmem.reshape(B * N, Tdim)
    lab = labels.reshape(B * N).astype(jnp.int32)
    partials = _stca_loss_sc(v2, lab)
    return jnp.sum(partials)


# speedup vs baseline: 315.2886x; 1.2315x over previous
import functools

import jax
import jax.numpy as jnp
from jax import lax
from jax.experimental import pallas as pl
from jax.experimental.pallas import tpu as pltpu
from jax.experimental.pallas import tpu_sc as plsc

THRESH = 1.0
GAP = 5
T = 512
L = 16
NC = 2
NS = 16
NW = NC * NS
ROWS = 32768
ROWS_PER_W = ROWS // NW
RB = 32
NBLK = ROWS_PER_W // RB
NCH = T // L
BIG = 1 << 22

_GDN = lax.GatherDimensionNumbers(
    offset_dims=(), collapsed_slice_dims=(0,), start_index_map=(0,)
)


def _vgather(x, idx):
    return lax.gather(
        x,
        idx[:, None],
        _GDN,
        slice_sizes=(1,),
        mode=lax.GatherScatterMode.PROMISE_IN_BOUNDS,
    )


def _splat_last(x):
    return _vgather(x, jnp.full((L,), L - 1, jnp.int32))


def _sc_body(v_hbm, lab_hbm, out_hbm, rows_v, posbuf, labels_v, accbuf, sems):
    wid = lax.axis_index("s") * NC + lax.axis_index("c")
    wstart = wid * ROWS_PER_W
    iota = lax.iota(jnp.int32, L)
    lane0 = iota == 0
    shl = jnp.maximum(iota - 1, 0)

    pltpu.sync_copy(lab_hbm.at[pl.ds(wstart, ROWS_PER_W)], labels_v)

    def row_body(r, rbase, d, acc):
        g = rbase + r
        labv = labels_v[pl.ds((g // L) * L, L)]
        lab_vec = _vgather(labv, jnp.full((L,), g % L, jnp.int32))
        lab_s = lab_vec[0]

        def miss_branch(_):
            @plsc.parallel_loop(
                0, NCH, carry=jnp.full((L,), -jnp.inf, jnp.float32), unroll=4
            )
            def vmx(j, c):
                return jnp.maximum(c, rows_v[d, r, pl.ds(j * L, L)])

            vmax_s = jnp.max(vmx)
            hit = lane0 & (lab_vec == 1) & (vmax_s < THRESH)
            return jnp.where(hit, THRESH - vmax_s, 0.0)

        def cluster_branch(_):

            @plsc.parallel_loop(
                0, NCH, carry=jnp.zeros((L,), jnp.int32), unroll=2
            )
            def n_splat(j, cnt):
                v_c = rows_v[d, r, pl.ds(j * L, L)]
                m = v_c >= THRESH
                s_c = plsc.cumsum(m.astype(jnp.int32)) + cnt
                gi = j * L + iota
                plsc.store_scatter(posbuf, [s_c - 1], gi, mask=m)
                return cnt + plsc.all_reduce_population_count(m)

            n_s = n_splat[0]

            def p2(j, c):
                si, prevlast, _, mink = c
                pos_c = posbuf[pl.ds(j * L, L)]
                gi = j * L + iota
                validm = gi < n_splat
                prev = jnp.where(lane0, prevlast, _vgather(pos_c, shl))
                gap = pos_c - prev
                newflag = (gap > GAP) & (gi > 0) & validm
                startval = jnp.where(newflag | (gi == 0), gi, 0)
                incl_loc = plsc.cummax(startval)
                excl = jnp.where(
                    lane0, si, jnp.maximum(_vgather(incl_loc, shl), si)
                )
                key = jnp.where(newflag, (gi - excl) * 1024 + prev, BIG)
                mink = jnp.minimum(mink, key)
                si = jnp.maximum(si, _splat_last(incl_loc))
                return si, _splat_last(pos_c), pos_c, mink

            nch = (n_s + (L - 1)) // L
            si0 = jnp.zeros((L,), jnp.int32)
            mink0 = jnp.full((L,), BIG, jnp.int32)
            si_f, _, lastp, mink_vec = lax.fori_loop(
                0, nch, p2, (si0, si0, si0, mink0)
            )

            lanes_last = jnp.bitwise_and(n_s - 1, L - 1)
            lp_s = _vgather(lastp, jnp.full((L,), lanes_last, jnp.int32))[0]
            mk = jnp.min(mink_vec)
            keyf = (n_s - si_f[0]) * 1024 + lp_s
            mk = jnp.minimum(mk, jnp.where(n_s > 0, keyf, BIG))
            tstar = jnp.bitwise_and(mk, 1023)

            vat = plsc.load_gather(
                rows_v,
                [
                    jnp.full((L,), d, jnp.int32),
                    jnp.full((L,), r, jnp.int32),
                    jnp.full((L,), tstar, jnp.int32),
                ],
            )
            return jnp.where(lane0 & (n_splat > 0), vat - THRESH, 0.0)

        contrib = lax.cond(lab_s == 0, cluster_branch, miss_branch, 0)
        return acc + contrib

    def _block_copy(b, d):
        return pltpu.make_async_copy(
            v_hbm.at[pl.ds(wstart + b * RB, RB)], rows_v.at[d], sems.at[d]
        )

    def blk_body(b, acc):
        d = jnp.bitwise_and(b, 1)
        _block_copy(b, d).wait()

        @pl.when(b + 1 < NBLK)
        def _():
            _block_copy(b + 1, 1 - d).start()

        rbase = b * RB

        def rloop(r, a):
            return row_body(r, rbase, d, a)

        return lax.fori_loop(0, RB, rloop, acc)

    _block_copy(0, 0).start()
    acc = lax.fori_loop(0, NBLK, blk_body, jnp.zeros((L,), jnp.float32))
    accbuf[...] = acc
    pltpu.sync_copy(accbuf, out_hbm.at[wid])


@functools.partial(
    pl.kernel,
    out_type=jax.ShapeDtypeStruct((NW, L), jnp.float32),
    compiler_params=pltpu.CompilerParams(needs_layout_passes=False),
    mesh=plsc.VectorSubcoreMesh(core_axis_name="c", subcore_axis_name="s"),
    scratch_types=[
        pltpu.VMEM((2, RB, T), jnp.float32),
        pltpu.VMEM((T,), jnp.int32),
        pltpu.VMEM((ROWS_PER_W,), jnp.int32),
        pltpu.VMEM((L,), jnp.float32),
        pltpu.SemaphoreType.DMA((2,)),
    ],
)
def _stca_loss_sc(v_hbm, lab_hbm, out_hbm, rows_v, posbuf, labels_v, accbuf, sems):
    _sc_body(v_hbm, lab_hbm, out_hbm, rows_v, posbuf, labels_v, accbuf, sems)


def kernel(vmem, labels):
    B, N, Tdim = vmem.shape
    v2 = vmem.reshape(B * N, Tdim)
    lab = labels.reshape(B * N).astype(jnp.int32)
    partials = _stca_loss_sc(v2, lab)
    return jnp.sum(partials)


# speedup vs baseline: 334.5656x; 1.0611x over previous
import functools

import jax
import jax.numpy as jnp
from jax import lax
from jax.experimental import pallas as pl
from jax.experimental.pallas import tpu as pltpu
from jax.experimental.pallas import tpu_sc as plsc

THRESH = 1.0
GAP = 5
T = 512
L = 16
NC = 2
NS = 16
NW = NC * NS
ROWS = 32768
ROWS_PER_W = ROWS // NW
RB = 32
NBLK = ROWS_PER_W // RB
NCH = T // L
BIG = 1 << 22

_GDN = lax.GatherDimensionNumbers(
    offset_dims=(), collapsed_slice_dims=(0,), start_index_map=(0,)
)


def _vgather(x, idx):
    return lax.gather(
        x,
        idx[:, None],
        _GDN,
        slice_sizes=(1,),
        mode=lax.GatherScatterMode.PROMISE_IN_BOUNDS,
    )


def _splat_last(x):
    return _vgather(x, jnp.full((L,), L - 1, jnp.int32))


def _sc_body(v_hbm, lab_hbm, out_hbm, rows_v, posbuf, labels_v, accbuf, sems):
    wid = lax.axis_index("s") * NC + lax.axis_index("c")
    wstart = wid * ROWS_PER_W
    iota = lax.iota(jnp.int32, L)
    lane0 = iota == 0
    shl = jnp.maximum(iota - 1, 0)

    pltpu.sync_copy(lab_hbm.at[pl.ds(wstart, ROWS_PER_W)], labels_v)

    def row_body(r, rbase, d, acc):
        g = rbase + r
        labv = labels_v[pl.ds((g // L) * L, L)]
        lab_vec = _vgather(labv, jnp.full((L,), g % L, jnp.int32))
        lab_s = lab_vec[0]

        def miss_branch(_):
            @plsc.parallel_loop(
                0, NCH, carry=jnp.full((L,), -jnp.inf, jnp.float32), unroll=8
            )
            def vmx(j, c):
                return jnp.maximum(c, rows_v[d, r, pl.ds(j * L, L)])

            vmax_s = jnp.max(vmx)
            hit = lane0 & (lab_vec == 1) & (vmax_s < THRESH)
            return jnp.where(hit, THRESH - vmax_s, 0.0)

        def cluster_branch(_):

            @plsc.parallel_loop(
                0, NCH, carry=jnp.zeros((L,), jnp.int32), unroll=4
            )
            def n_splat(j, cnt):
                v_c = rows_v[d, r, pl.ds(j * L, L)]
                m = v_c >= THRESH
                s_c = plsc.cumsum(m.astype(jnp.int32)) + cnt
                gi = j * L + iota
                plsc.store_scatter(posbuf, [s_c - 1], gi, mask=m)
                return cnt + plsc.all_reduce_population_count(m)

            n_s = n_splat[0]

            nch = (n_s + (L - 1)) // L
            si0 = jnp.zeros((L,), jnp.int32)
            mink0 = jnp.full((L,), BIG, jnp.int32)

            @plsc.parallel_loop(0, nch, carry=(si0, si0, si0, mink0))
            def p2_out(j, c):
                si, prevlast, _, mink = c
                pos_c = posbuf[pl.ds(j * L, L)]
                gi = j * L + iota
                validm = gi < n_splat
                prev = jnp.where(lane0, prevlast, _vgather(pos_c, shl))
                gap = pos_c - prev
                newflag = (gap > GAP) & (gi > 0) & validm
                startval = jnp.where(newflag | (gi == 0), gi, 0)
                incl_loc = plsc.cummax(startval)
                excl = jnp.where(
                    lane0, si, jnp.maximum(_vgather(incl_loc, shl), si)
                )
                key = jnp.where(newflag, (gi - excl) * 1024 + prev, BIG)
                mink = jnp.minimum(mink, key)
                si = jnp.maximum(si, _splat_last(incl_loc))
                return si, _splat_last(pos_c), pos_c, mink

            si_f, _, lastp, mink_vec = p2_out

            lanes_last = jnp.bitwise_and(n_s - 1, L - 1)
            lp_s = _vgather(lastp, jnp.full((L,), lanes_last, jnp.int32))[0]
            mk = jnp.min(mink_vec)
            keyf = (n_s - si_f[0]) * 1024 + lp_s
            mk = jnp.minimum(mk, jnp.where(n_s > 0, keyf, BIG))
            tstar = jnp.bitwise_and(mk, 1023)

            vat = plsc.load_gather(
                rows_v,
                [
                    jnp.full((L,), d, jnp.int32),
                    jnp.full((L,), r, jnp.int32),
                    jnp.full((L,), tstar, jnp.int32),
                ],
            )
            return jnp.where(lane0 & (n_splat > 0), vat - THRESH, 0.0)

        contrib = lax.cond(lab_s == 0, cluster_branch, miss_branch, 0)
        return acc + contrib

    def _block_copy(b, d):
        return pltpu.make_async_copy(
            v_hbm.at[pl.ds(wstart + b * RB, RB)], rows_v.at[d], sems.at[d]
        )

    def blk_body(b, acc):
        d = jnp.bitwise_and(b, 1)
        _block_copy(b, d).wait()

        @pl.when(b + 1 < NBLK)
        def _():
            _block_copy(b + 1, 1 - d).start()

        rbase = b * RB

        def rloop(r, a):
            return row_body(r, rbase, d, a)

        return lax.fori_loop(0, RB, rloop, acc)

    _block_copy(0, 0).start()
    acc = lax.fori_loop(0, NBLK, blk_body, jnp.zeros((L,), jnp.float32))
    accbuf[...] = acc
    pltpu.sync_copy(accbuf, out_hbm.at[wid])


@functools.partial(
    pl.kernel,
    out_type=jax.ShapeDtypeStruct((NW, L), jnp.float32),
    compiler_params=pltpu.CompilerParams(needs_layout_passes=False),
    mesh=plsc.VectorSubcoreMesh(core_axis_name="c", subcore_axis_name="s"),
    scratch_types=[
        pltpu.VMEM((2, RB, T), jnp.float32),
        pltpu.VMEM((T,), jnp.int32),
        pltpu.VMEM((ROWS_PER_W,), jnp.int32),
        pltpu.VMEM((L,), jnp.float32),
        pltpu.SemaphoreType.DMA((2,)),
    ],
)
def _stca_loss_sc(v_hbm, lab_hbm, out_hbm, rows_v, posbuf, labels_v, accbuf, sems):
    _sc_body(v_hbm, lab_hbm, out_hbm, rows_v, posbuf, labels_v, accbuf, sems)


def kernel(vmem, labels):
    B, N, Tdim = vmem.shape
    v2 = vmem.reshape(B * N, Tdim)
    lab = labels.reshape(B * N).astype(jnp.int32)
    partials = _stca_loss_sc(v2, lab)
    return jnp.sum(partials)
